# baseline (speedup 1.0000x reference)
import functools

import jax
import jax.numpy as jnp
from jax.experimental import pallas as pl

_EMBED = 256
_S = 8
_ROWS_PER_BLOCK = 1024


def _embed_body(xv, xd, x0, x1, x2, Wv, Wd, Wp, bv, bd, bp, out):
    dn = (((1,), (1,)), ((), ()))
    acc = jax.lax.dot_general(xv[...], Wv[...], dn,
                              preferred_element_type=jnp.float32)
    acc = acc + jax.lax.dot_general(xd[...], Wd[...], dn,
                                    preferred_element_type=jnp.float32)
    for a, xa in enumerate((x0, x1, x2)):
        acc = acc + jax.lax.dot_general(xa[...], Wp[a], dn,
                                        preferred_element_type=jnp.float32)
    bias = bv[...] + bd[...] + jnp.sum(bp[...], axis=0, keepdims=True)
    out[...] = acc + bias


@jax.jit
def kernel(value, depth, pos, Wv, bv, Wd, bd, Wp, bp):
    B, L = value.shape
    T = L // _S
    N = B * T

    xv = value.astype(jnp.float32).reshape(N, _S)
    xd = depth.astype(jnp.float32).reshape(N, _S)
    p = pos.astype(jnp.float32)
    x0 = p[:, :, 0].reshape(N, _S)
    x1 = p[:, :, 1].reshape(N, _S)
    x2 = p[:, :, 2].reshape(N, _S)

    bv2 = bv.reshape(1, _EMBED)
    bd2 = bd.reshape(1, _EMBED)

    R = _ROWS_PER_BLOCK
    grid = (N // R,)

    row_spec = pl.BlockSpec((R, _S), lambda i: (i, 0))
    w_spec = pl.BlockSpec((_EMBED, _S), lambda i: (0, 0))
    wp_spec = pl.BlockSpec((3, _EMBED, _S), lambda i: (0, 0, 0))
    b_spec = pl.BlockSpec((1, _EMBED), lambda i: (0, 0))
    bp_spec = pl.BlockSpec((3, _EMBED), lambda i: (0, 0))

    out = pl.pallas_call(
        _embed_body,
        grid=grid,
        in_specs=[row_spec, row_spec, row_spec, row_spec, row_spec,
                  w_spec, w_spec, wp_spec, b_spec, b_spec, bp_spec],
        out_specs=pl.BlockSpec((R, _EMBED), lambda i: (i, 0)),
        out_shape=jax.ShapeDtypeStruct((N, _EMBED), jnp.float32),
    )(xv, xd, x0, x1, x2, Wv, Wd, Wp, bv2, bd2, bp)

    return out.reshape(B, T, _EMBED)
